# baseline (speedup 1.0000x reference)
import jax
import jax.numpy as jnp
from jax.experimental import pallas as pl
from jax.experimental.pallas import tpu as pltpu

_TS = 2048
_G = 64


def _router_body(x_ref, gt_ref, w_ref, b_ref, sl_ref, out_ref, sem, gsem, *, n_per_b, s_out):
    k = pl.program_id(0)
    bi = k // n_per_b
    i = k % n_per_b

    cp = pltpu.make_async_copy(
        x_ref.at[0],
        out_ref.at[pl.ds(bi * s_out + _G + i * _TS, _TS), :],
        sem,
    )
    cp.start()

    @pl.when(i == 0)
    def _():
        gcp = pltpu.make_async_copy(
            gt_ref, out_ref.at[pl.ds(bi * s_out, _G), :], gsem
        )
        gcp.start()
        gcp.wait()

    xb = x_ref[0]
    r = jnp.dot(xb, w_ref[...], preferred_element_type=jnp.float32)
    r = r + b_ref[...]
    lane = jax.lax.broadcasted_iota(jnp.int32, r.shape, 1)
    sl_ref[0] = jnp.where(lane == 0, jax.nn.sigmoid(r), r)

    cp.wait()


def kernel(x, global_tokens, anchor_w, anchor_b, seg_w, seg_b):
    b, s, h = x.shape
    g = global_tokens.shape[0]
    n_per_b = s // _TS
    n = b * n_per_b
    s_out = g + s

    w = jnp.concatenate([anchor_w, seg_w], axis=1)
    bias = jnp.stack([anchor_b[0], seg_b[0]]).reshape(1, 2)
    xf = x.reshape(b * s, h)

    import functools
    body = functools.partial(_router_body, n_per_b=n_per_b, s_out=s_out)

    sl, out = pl.pallas_call(
        body,
        grid=(n,),
        in_specs=[
            pl.BlockSpec((1, _TS, h), lambda k: (k, 0, 0)),
            pl.BlockSpec((g, h), lambda k: (0, 0)),
            pl.BlockSpec((h, 2), lambda k: (0, 0)),
            pl.BlockSpec((1, 2), lambda k: (0, 0)),
        ],
        out_specs=[
            pl.BlockSpec((1, _TS, 2), lambda k: (k, 0, 0)),
            pl.BlockSpec(memory_space=pltpu.HBM),
        ],
        out_shape=[
            jax.ShapeDtypeStruct((n, _TS, 2), jnp.float32),
            jax.ShapeDtypeStruct((b * s_out, h), jnp.float32),
        ],
        scratch_shapes=[pltpu.SemaphoreType.DMA, pltpu.SemaphoreType.DMA],
    )(xf.reshape(n, _TS, h), global_tokens, w, bias)

    sl = sl.reshape(b, s, 2)
    out = out.reshape(b, s_out, h)
    anchor_scores = sl[:, :, 0]
    segment_logits = sl[:, :, 1]
    mask_row = jnp.arange(s_out, dtype=jnp.int32) < g
    global_mask = jnp.broadcast_to(mask_row[None, :], (b, s_out))
    return (out, global_mask, anchor_scores, segment_logits)


# speedup vs baseline: 1.1985x; 1.1985x over previous
import jax
import jax.numpy as jnp
from jax.experimental import pallas as pl
from jax.experimental.pallas import tpu as pltpu

_TS = 2048
_G = 64


def _router_body(x_ref, gt_ref, w_ref, b_ref, sl_ref, out_ref, sem, gsem):
    bi = pl.program_id(0)
    i = pl.program_id(1)

    cp = pltpu.make_async_copy(
        x_ref.at[0],
        out_ref.at[bi, pl.ds(_G + i * _TS, _TS), :],
        sem,
    )
    cp.start()

    @pl.when(i == 0)
    def _():
        gcp = pltpu.make_async_copy(gt_ref, out_ref.at[bi, pl.ds(0, _G), :], gsem)
        gcp.start()
        gcp.wait()

    xb = x_ref[0]
    r = jnp.dot(xb, w_ref[...], preferred_element_type=jnp.float32)
    r = r + b_ref[...]
    lane = jax.lax.broadcasted_iota(jnp.int32, r.shape, 1)
    sl_ref[0] = jnp.where(lane == 0, jax.nn.sigmoid(r), r)

    cp.wait()


def kernel(x, global_tokens, anchor_w, anchor_b, seg_w, seg_b):
    b, s, h = x.shape
    g = global_tokens.shape[0]
    n = s // _TS

    w = jnp.concatenate([anchor_w, seg_w], axis=1)
    bias = jnp.stack([anchor_b[0], seg_b[0]]).reshape(1, 2)

    sl, out = pl.pallas_call(
        _router_body,
        grid=(b, n),
        in_specs=[
            pl.BlockSpec((1, _TS, h), lambda i, j: (i, j, 0)),
            pl.BlockSpec((g, h), lambda i, j: (0, 0)),
            pl.BlockSpec((h, 2), lambda i, j: (0, 0)),
            pl.BlockSpec((1, 2), lambda i, j: (0, 0)),
        ],
        out_specs=[
            pl.BlockSpec((1, _TS, 2), lambda i, j: (i, j, 0)),
            pl.BlockSpec(memory_space=pltpu.HBM),
        ],
        out_shape=[
            jax.ShapeDtypeStruct((b, s, 2), jnp.float32),
            jax.ShapeDtypeStruct((b, g + s, h), jnp.float32),
        ],
        scratch_shapes=[pltpu.SemaphoreType.DMA, pltpu.SemaphoreType.DMA],
    )(x, global_tokens, w, bias)

    anchor_scores = sl[:, :, 0]
    segment_logits = sl[:, :, 1]
    mask_row = jnp.arange(g + s, dtype=jnp.int32) < g
    global_mask = jnp.broadcast_to(mask_row[None, :], (b, g + s))
    return (out, global_mask, anchor_scores, segment_logits)
